# Initial kernel scaffold; baseline (speedup 1.0000x reference)
import jax
import jax.numpy as jnp
from jax.experimental import pallas as pl


def kernel(x, edge_index, edge_attr, params):
    raise NotImplementedError("write your pallas kernel here")


# speedup vs baseline: 2.1167x; 2.1167x over previous
import functools

import jax
import jax.numpy as jnp
from jax import lax
from jax.experimental import pallas as pl
from jax.experimental.pallas import tpu as pltpu
from jax.experimental.pallas import tpu_sc as plsc

F32 = jnp.float32

_N = 10000
_E = 320000
_H = 128

_NSC = 2
_NT = 16
_EPT = _E // (_NSC * _NT)
_BE = 80
_NCHUNK = _EPT // _BE
_RPT = 624
_NTAIL = _N - _NT * _RPT


def _mm_bias(x, w, b, *, relu, block_rows):
    m, k = x.shape
    n = w.shape[1]

    def body(x_ref, w_ref, b_ref, o_ref):
        y = jnp.dot(x_ref[...], w_ref[...], preferred_element_type=F32)
        y = y + b_ref[...]
        if relu:
            y = jnp.maximum(y, 0.0)
        o_ref[...] = y

    return pl.pallas_call(
        body,
        grid=(m // block_rows,),
        in_specs=[
            pl.BlockSpec((block_rows, k), lambda i: (i, 0)),
            pl.BlockSpec((k, n), lambda i: (0, 0)),
            pl.BlockSpec((1, n), lambda i: (0, 0)),
        ],
        out_specs=pl.BlockSpec((block_rows, n), lambda i: (i, 0)),
        out_shape=jax.ShapeDtypeStruct((m, n), F32),
    )(x, w, b.reshape(1, -1))


def _edge_pre(ea, wcat, bcat):
    be = 2000

    def body(ea_ref, w_ref, b_ref, o0, o1, o2, o3):
        y = jnp.dot(ea_ref[...], w_ref[...], preferred_element_type=F32)
        y = y + b_ref[...]
        o0[...] = y[:, 0:128]
        o1[...] = y[:, 128:256]
        o2[...] = y[:, 256:384]
        o3[...] = y[:, 384:512]

    return pl.pallas_call(
        body,
        grid=(_E // be,),
        in_specs=[
            pl.BlockSpec((be, 16), lambda i: (i, 0)),
            pl.BlockSpec((16, 512), lambda i: (0, 0)),
            pl.BlockSpec((1, 512), lambda i: (0, 0)),
        ],
        out_specs=[pl.BlockSpec((be, 128), lambda i: (i, 0))] * 4,
        out_shape=[jax.ShapeDtypeStruct((_E, 128), F32)] * 4,
    )(ea, wcat, bcat.reshape(1, -1))


def _node_mlp(h, p0, p1, w1, b1, w2, b2, gamma, beta):
    bn = 1000
    nb = _N // bn

    def body(h_ref, p0_ref, p1_ref, w1_ref, b1_ref, w2_ref, b2_ref,
             g_ref, be_ref, o_ref, y_scr, stat_scr):
        i = pl.program_id(0)

        z = h_ref[...] + p0_ref[...] + p1_ref[...]
        y = jnp.dot(z, w1_ref[...], preferred_element_type=F32) + b1_ref[...]
        y = jnp.maximum(y, 0.0)
        y = jnp.dot(y, w2_ref[...], preferred_element_type=F32) + b2_ref[...]
        y_scr[pl.ds(i * bn, bn), :] = y
        s = jnp.sum(y, axis=0, keepdims=True)
        ss = jnp.sum(y * y, axis=0, keepdims=True)

        @pl.when(i == 0)
        def _():
            stat_scr[0:1, :] = s
            stat_scr[1:2, :] = ss

        @pl.when(i > 0)
        def _():
            stat_scr[0:1, :] = stat_scr[0:1, :] + s
            stat_scr[1:2, :] = stat_scr[1:2, :] + ss

        @pl.when(i == nb - 1)
        def _():
            mean = stat_scr[0:1, :] * (1.0 / _N)
            var = stat_scr[1:2, :] * (1.0 / _N) - mean * mean
            scale = lax.rsqrt(var + 1e-5) * g_ref[...]
            shift = be_ref[...] - mean * scale
            for j in range(nb):
                yj = y_scr[pl.ds(j * bn, bn), :]
                o_ref[pl.ds(j * bn, bn), :] = jnp.maximum(
                    yj * scale + shift, 0.0)

    return pl.pallas_call(
        body,
        grid=(nb,),
        in_specs=[
            pl.BlockSpec((bn, _H), lambda i: (i, 0)),
            pl.BlockSpec((bn, _H), lambda i: (i, 0)),
            pl.BlockSpec((bn, _H), lambda i: (i, 0)),
            pl.BlockSpec((_H, 2 * _H), lambda i: (0, 0)),
            pl.BlockSpec((1, 2 * _H), lambda i: (0, 0)),
            pl.BlockSpec((2 * _H, _H), lambda i: (0, 0)),
            pl.BlockSpec((1, _H), lambda i: (0, 0)),
            pl.BlockSpec((1, _H), lambda i: (0, 0)),
            pl.BlockSpec((1, _H), lambda i: (0, 0)),
        ],
        out_specs=pl.BlockSpec((_N, _H), lambda i: (0, 0)),
        out_shape=jax.ShapeDtypeStruct((_N, _H), F32),
        scratch_shapes=[
            pltpu.VMEM((_N, _H), F32),
            pltpu.VMEM((8, _H), F32),
        ],
    )(h, p0, p1, w1, b1.reshape(1, -1), w2, b2.reshape(1, -1),
      gamma.reshape(1, -1), beta.reshape(1, -1))


def _ab_tables(h, wab):
    bn = 1000

    def body(h_ref, w_ref, oa, ob):
        y = jnp.dot(h_ref[...], w_ref[...], preferred_element_type=F32)
        oa[...] = y[:, 0:128]
        ob[...] = y[:, 128:256]

    return pl.pallas_call(
        body,
        grid=(_N // bn,),
        in_specs=[
            pl.BlockSpec((bn, _H), lambda i: (i, 0)),
            pl.BlockSpec((_H, 2 * _H), lambda i: (0, 0)),
        ],
        out_specs=[pl.BlockSpec((bn, _H), lambda i: (i, 0))] * 2,
        out_shape=[jax.ShapeDtypeStruct((_N, _H), F32)] * 2,
    )(h, wab)


def _head_mlp(t, w2, b2, w3, b3):
    be = 2000
    c = w3.shape[1]

    def body(t_ref, w2_ref, b2_ref, w3_ref, b3_ref, o_ref):
        y = jnp.dot(t_ref[...], w2_ref[...], preferred_element_type=F32)
        y = jnp.maximum(y + b2_ref[...], 0.0)
        o_ref[...] = jnp.dot(y, w3_ref[...], preferred_element_type=F32) + b3_ref[...]

    return pl.pallas_call(
        body,
        grid=(_E // be,),
        in_specs=[
            pl.BlockSpec((be, _H), lambda i: (i, 0)),
            pl.BlockSpec((_H, 64), lambda i: (0, 0)),
            pl.BlockSpec((1, 64), lambda i: (0, 0)),
            pl.BlockSpec((64, c), lambda i: (0, 0)),
            pl.BlockSpec((1, c), lambda i: (0, 0)),
        ],
        out_specs=pl.BlockSpec((be, c), lambda i: (i, 0)),
        out_shape=jax.ShapeDtypeStruct((_E, c), F32),
    )(t, w2, b2.reshape(1, -1), w3, b3.reshape(1, -1))


@functools.lru_cache(maxsize=None)
def _sc_mesh():
    return plsc.VectorSubcoreMesh(core_axis_name="c", subcore_axis_name="s",
                                  num_cores=_NSC, num_subcores=_NT)


def _relu_add_rows(msg, extra, nrows):
    def row(r, _):
        for cc in range(_H // 16):
            sl = pl.ds(cc * 16, 16)
            v = msg[r, sl]
            for ex in extra:
                v = v + ex[r, sl]
            msg[r, sl] = jnp.maximum(v, 0.0)
        return 0

    lax.fori_loop(0, nrows, row, 0)


@functools.lru_cache(maxsize=None)
def _make_sc_layer():
    return functools.partial(
        pl.kernel,
        out_type=jax.ShapeDtypeStruct((_NSC, _N, _H), F32),
        mesh=_sc_mesh(),
        scratch_types=[
            pltpu.VMEM((_BE,), jnp.int32),
            pltpu.VMEM((_BE,), jnp.int32),
            pltpu.VMEM((_BE, _H), F32),
            pltpu.VMEM((_BE, _H), F32),
            pltpu.VMEM((104, _H), F32),
            pltpu.SemaphoreType.DMA,
            pltpu.VMEM_SHARED((_N, _H), F32),
        ],
    )(_sc_layer_body)


def _sc_layer_body(h_hbm, e_hbm, src_hbm, dst_hbm, out_hbm,
                   idx_s, idx_d, msg, rows, zbuf, sem, aggr):
    cid = lax.axis_index("c")
    sid = lax.axis_index("s")

    zero16 = jnp.zeros((16,), F32)

    def zrow(r, _):
        for cc in range(_H // 16):
            zbuf[r, pl.ds(cc * 16, 16)] = zero16
        return 0

    lax.fori_loop(0, 104, zrow, 0)
    row0 = sid * _RPT
    for j in range(_RPT // 104):
        pltpu.sync_copy(zbuf, aggr.at[pl.ds(row0 + j * 104, 104)])

    @pl.when(sid == _NT - 1)
    def _():
        pltpu.sync_copy(zbuf.at[pl.ds(0, _NTAIL)],
                        aggr.at[pl.ds(_NT * _RPT, _NTAIL)])

    plsc.subcore_barrier()

    ebase = cid * (_E // _NSC) + sid * _EPT

    def chunk(ci, _):
        base = ebase + ci * _BE
        pltpu.sync_copy(src_hbm.at[pl.ds(base, _BE)], idx_s)
        pltpu.sync_copy(dst_hbm.at[pl.ds(base, _BE)], idx_d)
        pltpu.sync_copy(e_hbm.at[pl.ds(base, _BE)], msg)
        pltpu.async_copy(h_hbm.at[idx_s], rows, sem).wait()
        _relu_add_rows(msg, (rows,), _BE)
        pltpu.sync_copy(msg, aggr.at[idx_d], add=True)
        return 0

    lax.fori_loop(0, _NCHUNK, chunk, 0)
    plsc.subcore_barrier()
    pltpu.sync_copy(aggr.at[pl.ds(row0, _RPT)],
                    out_hbm.at[cid, pl.ds(row0, _RPT)])

    @pl.when(sid == _NT - 1)
    def _():
        pltpu.sync_copy(aggr.at[pl.ds(_NT * _RPT, _NTAIL)],
                        out_hbm.at[cid, pl.ds(_NT * _RPT, _NTAIL)])


@functools.lru_cache(maxsize=None)
def _make_sc_head():
    return functools.partial(
        pl.kernel,
        out_type=jax.ShapeDtypeStruct((_E, _H), F32),
        mesh=_sc_mesh(),
        scratch_types=[
            pltpu.VMEM((_BE,), jnp.int32),
            pltpu.VMEM((_BE,), jnp.int32),
            pltpu.VMEM((_BE, _H), F32),
            pltpu.VMEM((_BE, _H), F32),
            pltpu.VMEM((_BE, _H), F32),
            pltpu.SemaphoreType.DMA,
        ],
    )(_sc_head_body)


def _sc_head_body(a_hbm, b_hbm, eh_hbm, src_hbm, dst_hbm, t_hbm,
                  idx_s, idx_d, msg, rows_a, rows_b, sem):
    cid = lax.axis_index("c")
    sid = lax.axis_index("s")
    ebase = cid * (_E // _NSC) + sid * _EPT

    def chunk(ci, _):
        base = ebase + ci * _BE
        pltpu.sync_copy(src_hbm.at[pl.ds(base, _BE)], idx_s)
        pltpu.sync_copy(dst_hbm.at[pl.ds(base, _BE)], idx_d)
        pltpu.sync_copy(eh_hbm.at[pl.ds(base, _BE)], msg)
        d1 = pltpu.async_copy(a_hbm.at[idx_s], rows_a, sem)
        d2 = pltpu.async_copy(b_hbm.at[idx_d], rows_b, sem)
        d1.wait()
        d2.wait()
        _relu_add_rows(msg, (rows_a, rows_b), _BE)
        pltpu.sync_copy(msg, t_hbm.at[pl.ds(base, _BE)])
        return 0

    lax.fori_loop(0, _NCHUNK, chunk, 0)


def kernel(x, edge_index, edge_attr, params):
    p = params
    layers = p['layers']
    src = edge_index[0]
    dst = edge_index[1]

    wcat = jnp.concatenate(
        [layers[0]['lin_w'], layers[1]['lin_w'], layers[2]['lin_w'],
         p['hw1'][2 * _H:]], axis=1)
    bcat = jnp.concatenate(
        [layers[0]['lin_b'], layers[1]['lin_b'], layers[2]['lin_b'], p['hb1']])
    e1, e2, e3, eh = _edge_pre(edge_attr, wcat, bcat)

    h = _mm_bias(x, p['in_w'], p['in_b'], relu=True, block_rows=1000)

    sc_layer = _make_sc_layer()
    for lp, e_l in zip(layers, (e1, e2, e3)):
        partials = sc_layer(h, e_l, src, dst)
        h = _node_mlp(h, partials[0], partials[1],
                      lp['w1'], lp['b1'], lp['w2'], lp['b2'],
                      lp['gamma'], lp['beta'])

    wab = jnp.concatenate([p['hw1'][:_H], p['hw1'][_H:2 * _H]], axis=1)
    a_tab, b_tab = _ab_tables(h, wab)
    t = _make_sc_head()(a_tab, b_tab, eh, src, dst)
    return _head_mlp(t, p['hw2'], p['hb2'], p['hw3'], p['hb3'])


# speedup vs baseline: 3.7274x; 1.7610x over previous
import functools

import jax
import jax.numpy as jnp
from jax import lax
from jax.experimental import pallas as pl
from jax.experimental.pallas import tpu as pltpu
from jax.experimental.pallas import tpu_sc as plsc

F32 = jnp.float32

_N = 10000
_E = 320000
_H = 128

_NSC = 2
_NT = 16
_EPT = _E // (_NSC * _NT)
_BE = 80
_NCHUNK = _EPT // _BE
_RPT = 624
_NTAIL = _N - _NT * _RPT


def _mm_bias(x, w, b, *, relu, block_rows):
    m, k = x.shape
    n = w.shape[1]

    def body(x_ref, w_ref, b_ref, o_ref):
        y = jnp.dot(x_ref[...], w_ref[...], preferred_element_type=F32)
        y = y + b_ref[...]
        if relu:
            y = jnp.maximum(y, 0.0)
        o_ref[...] = y

    return pl.pallas_call(
        body,
        grid=(m // block_rows,),
        in_specs=[
            pl.BlockSpec((block_rows, k), lambda i: (i, 0)),
            pl.BlockSpec((k, n), lambda i: (0, 0)),
            pl.BlockSpec((1, n), lambda i: (0, 0)),
        ],
        out_specs=pl.BlockSpec((block_rows, n), lambda i: (i, 0)),
        out_shape=jax.ShapeDtypeStruct((m, n), F32),
    )(x, w, b.reshape(1, -1))


def _edge_pre(ea, wcat, bcat):
    be = 2000

    def body(ea_ref, w_ref, b_ref, o0, o1, o2, o3):
        y = jnp.dot(ea_ref[...], w_ref[...], preferred_element_type=F32)
        y = y + b_ref[...]
        o0[...] = y[:, 0:128]
        o1[...] = y[:, 128:256]
        o2[...] = y[:, 256:384]
        o3[...] = y[:, 384:512]

    return pl.pallas_call(
        body,
        grid=(_E // be,),
        in_specs=[
            pl.BlockSpec((be, 16), lambda i: (i, 0)),
            pl.BlockSpec((16, 512), lambda i: (0, 0)),
            pl.BlockSpec((1, 512), lambda i: (0, 0)),
        ],
        out_specs=[pl.BlockSpec((be, 128), lambda i: (i, 0))] * 4,
        out_shape=[jax.ShapeDtypeStruct((_E, 128), F32)] * 4,
    )(ea, wcat, bcat.reshape(1, -1))


def _node_mlp(h, p0, p1, w1, b1, w2, b2, gamma, beta):
    bn = 1000
    nb = _N // bn

    def body(h_ref, p0_ref, p1_ref, w1_ref, b1_ref, w2_ref, b2_ref,
             g_ref, be_ref, o_ref, y_scr, stat_scr):
        i = pl.program_id(0)

        z = h_ref[...] + p0_ref[...] + p1_ref[...]
        y = jnp.dot(z, w1_ref[...], preferred_element_type=F32) + b1_ref[...]
        y = jnp.maximum(y, 0.0)
        y = jnp.dot(y, w2_ref[...], preferred_element_type=F32) + b2_ref[...]
        y_scr[pl.ds(i * bn, bn), :] = y
        s = jnp.sum(y, axis=0, keepdims=True)
        ss = jnp.sum(y * y, axis=0, keepdims=True)

        @pl.when(i == 0)
        def _():
            stat_scr[0:1, :] = s
            stat_scr[1:2, :] = ss

        @pl.when(i > 0)
        def _():
            stat_scr[0:1, :] = stat_scr[0:1, :] + s
            stat_scr[1:2, :] = stat_scr[1:2, :] + ss

        @pl.when(i == nb - 1)
        def _():
            mean = stat_scr[0:1, :] * (1.0 / _N)
            var = stat_scr[1:2, :] * (1.0 / _N) - mean * mean
            scale = lax.rsqrt(var + 1e-5) * g_ref[...]
            shift = be_ref[...] - mean * scale
            for j in range(nb):
                yj = y_scr[pl.ds(j * bn, bn), :]
                o_ref[pl.ds(j * bn, bn), :] = jnp.maximum(
                    yj * scale + shift, 0.0)

    return pl.pallas_call(
        body,
        grid=(nb,),
        in_specs=[
            pl.BlockSpec((bn, _H), lambda i: (i, 0)),
            pl.BlockSpec((bn, _H), lambda i: (i, 0)),
            pl.BlockSpec((bn, _H), lambda i: (i, 0)),
            pl.BlockSpec((_H, 2 * _H), lambda i: (0, 0)),
            pl.BlockSpec((1, 2 * _H), lambda i: (0, 0)),
            pl.BlockSpec((2 * _H, _H), lambda i: (0, 0)),
            pl.BlockSpec((1, _H), lambda i: (0, 0)),
            pl.BlockSpec((1, _H), lambda i: (0, 0)),
            pl.BlockSpec((1, _H), lambda i: (0, 0)),
        ],
        out_specs=pl.BlockSpec((_N, _H), lambda i: (0, 0)),
        out_shape=jax.ShapeDtypeStruct((_N, _H), F32),
        scratch_shapes=[
            pltpu.VMEM((_N, _H), F32),
            pltpu.VMEM((8, _H), F32),
        ],
    )(h, p0, p1, w1, b1.reshape(1, -1), w2, b2.reshape(1, -1),
      gamma.reshape(1, -1), beta.reshape(1, -1))


def _ab_tables(h, wab):
    bn = 1000

    def body(h_ref, w_ref, oa, ob):
        y = jnp.dot(h_ref[...], w_ref[...], preferred_element_type=F32)
        oa[...] = y[:, 0:128]
        ob[...] = y[:, 128:256]

    return pl.pallas_call(
        body,
        grid=(_N // bn,),
        in_specs=[
            pl.BlockSpec((bn, _H), lambda i: (i, 0)),
            pl.BlockSpec((_H, 2 * _H), lambda i: (0, 0)),
        ],
        out_specs=[pl.BlockSpec((bn, _H), lambda i: (i, 0))] * 2,
        out_shape=[jax.ShapeDtypeStruct((_N, _H), F32)] * 2,
    )(h, wab)


def _head_mlp(t, w2, b2, w3, b3):
    be = 2000
    c = w3.shape[1]

    def body(t_ref, w2_ref, b2_ref, w3_ref, b3_ref, o_ref):
        y = jnp.dot(t_ref[...], w2_ref[...], preferred_element_type=F32)
        y = jnp.maximum(y + b2_ref[...], 0.0)
        o_ref[...] = jnp.dot(y, w3_ref[...], preferred_element_type=F32) + b3_ref[...]

    return pl.pallas_call(
        body,
        grid=(_E // be,),
        in_specs=[
            pl.BlockSpec((be, _H), lambda i: (i, 0)),
            pl.BlockSpec((_H, 64), lambda i: (0, 0)),
            pl.BlockSpec((1, 64), lambda i: (0, 0)),
            pl.BlockSpec((64, c), lambda i: (0, 0)),
            pl.BlockSpec((1, c), lambda i: (0, 0)),
        ],
        out_specs=pl.BlockSpec((be, c), lambda i: (i, 0)),
        out_shape=jax.ShapeDtypeStruct((_E, c), F32),
    )(t, w2, b2.reshape(1, -1), w3, b3.reshape(1, -1))


@functools.lru_cache(maxsize=None)
def _sc_mesh():
    return plsc.VectorSubcoreMesh(core_axis_name="c", subcore_axis_name="s",
                                  num_cores=_NSC, num_subcores=_NT)


def _relu_add_rows(msg, extra, nrows):
    def row(r, _):
        for cc in range(_H // 16):
            sl = pl.ds(cc * 16, 16)
            v = msg[r, sl]
            for ex in extra:
                v = v + ex[r, sl]
            msg[r, sl] = jnp.maximum(v, 0.0)
        return 0

    lax.fori_loop(0, nrows, row, 0)


def _when(cond, fn):
    if isinstance(cond, bool):
        if cond:
            fn()
    else:
        pl.when(cond)(fn)


@functools.lru_cache(maxsize=None)
def _make_sc_layer():
    return functools.partial(
        pl.kernel,
        out_type=jax.ShapeDtypeStruct((_NSC, _N, _H), F32),
        mesh=_sc_mesh(),
        scratch_types=[
            pltpu.VMEM((2, _BE), jnp.int32),
            pltpu.VMEM((2, _BE), jnp.int32),
            pltpu.VMEM((_BE, _H), F32),
            pltpu.VMEM((_BE, _H), F32),
            pltpu.VMEM((_BE, _H), F32),
            pltpu.VMEM((_BE, _H), F32),
            pltpu.VMEM((48, _H), F32),
            pltpu.SemaphoreType.DMA,
            pltpu.SemaphoreType.DMA,
            pltpu.SemaphoreType.DMA,
            pltpu.SemaphoreType.DMA,
            pltpu.SemaphoreType.DMA,
            pltpu.SemaphoreType.DMA,
            pltpu.SemaphoreType.DMA,
            pltpu.SemaphoreType.DMA,
            pltpu.VMEM_SHARED((_N, _H), F32),
        ],
    )(_sc_layer_body)


def _sc_layer_body(h_hbm, e_hbm, src_hbm, dst_hbm, out_hbm,
                   idx_s2, idx_d2, msg0, msg1, rows0, rows1, zbuf,
                   si0, si1, se0, se1, sg0, sg1, ss0, ss1, aggr):
    idx_s = (idx_s2.at[0], idx_s2.at[1])
    idx_d = (idx_d2.at[0], idx_d2.at[1])
    msg = (msg0, msg1)
    rows = (rows0, rows1)
    sem_i = (si0, si1)
    sem_e = (se0, se1)
    sem_g = (sg0, sg1)
    sem_s = (ss0, ss1)
    cid = lax.axis_index("c")
    sid = lax.axis_index("s")

    zero16 = jnp.zeros((16,), F32)

    def zrow(r, _):
        for cc in range(_H // 16):
            zbuf[r, pl.ds(cc * 16, 16)] = zero16
        return 0

    lax.fori_loop(0, 48, zrow, 0)
    row0 = sid * _RPT
    for j in range(_RPT // 48):
        pltpu.sync_copy(zbuf, aggr.at[pl.ds(row0 + j * 48, 48)])

    @pl.when(sid == _NT - 1)
    def _():
        pltpu.sync_copy(zbuf.at[pl.ds(0, _NTAIL)],
                        aggr.at[pl.ds(_NT * _RPT, _NTAIL)])

    plsc.subcore_barrier()

    ebase = cid * (_E // _NSC) + sid * _EPT

    def issue(g, k):
        base = ebase + g * _BE
        pltpu.async_copy(src_hbm.at[pl.ds(base, _BE)], idx_s[k], sem_i[k])
        pltpu.async_copy(dst_hbm.at[pl.ds(base, _BE)], idx_d[k], sem_i[k])
        pltpu.async_copy(e_hbm.at[pl.ds(base, _BE)], msg[k], sem_e[k])
        pltpu.make_async_copy(src_hbm.at[pl.ds(base, _BE)], idx_s[k],
                              sem_i[k]).wait()
        pltpu.async_copy(h_hbm.at[idx_s[k]], rows[k], sem_g[k])

    def stage(g, b, do_drain, do_issue_next):
        kc, kn = b, 1 - b
        base = ebase + g * _BE

        _when(do_drain, lambda: pltpu.make_async_copy(
            msg[kn], aggr.at[idx_d[kn]], sem_s[kn]).wait())
        _when(do_issue_next, lambda: issue(g + 1, kn))
        pltpu.make_async_copy(e_hbm.at[pl.ds(base, _BE)], msg[kc],
                              sem_e[kc]).wait()
        pltpu.make_async_copy(h_hbm.at[idx_s[kc]], rows[kc], sem_g[kc]).wait()
        _relu_add_rows(msg[kc], (rows[kc],), _BE)
        pltpu.make_async_copy(dst_hbm.at[pl.ds(base, _BE)], idx_d[kc],
                              sem_i[kc]).wait()
        pltpu.async_copy(msg[kc], aggr.at[idx_d[kc]], sem_s[kc], add=True)

    issue(0, 0)

    def pair(gg, _):
        g0 = gg * 2
        stage(g0, 0, g0 > 0, True)
        stage(g0 + 1, 1, True, True)
        return 0

    lax.fori_loop(0, _NCHUNK // 2, pair, 0)
    stage(_NCHUNK - 1, 0, True, False)
    pltpu.make_async_copy(msg[0], aggr.at[idx_d[0]], sem_s[0]).wait()
    plsc.subcore_barrier()
    pltpu.sync_copy(aggr.at[pl.ds(row0, _RPT)],
                    out_hbm.at[cid, pl.ds(row0, _RPT)])

    @pl.when(sid == _NT - 1)
    def _():
        pltpu.sync_copy(aggr.at[pl.ds(_NT * _RPT, _NTAIL)],
                        out_hbm.at[cid, pl.ds(_NT * _RPT, _NTAIL)])


@functools.lru_cache(maxsize=None)
def _make_sc_head():
    return functools.partial(
        pl.kernel,
        out_type=jax.ShapeDtypeStruct((_E, _H), F32),
        mesh=_sc_mesh(),
        scratch_types=[
            pltpu.VMEM((2, _BE), jnp.int32),
            pltpu.VMEM((2, _BE), jnp.int32),
            pltpu.VMEM((_BE, _H), F32),
            pltpu.VMEM((_BE, _H), F32),
            pltpu.VMEM((_BE, _H), F32),
            pltpu.VMEM((_BE, _H), F32),
            pltpu.VMEM((_BE, _H), F32),
            pltpu.VMEM((_BE, _H), F32),
            pltpu.SemaphoreType.DMA,
            pltpu.SemaphoreType.DMA,
            pltpu.SemaphoreType.DMA,
            pltpu.SemaphoreType.DMA,
            pltpu.SemaphoreType.DMA,
            pltpu.SemaphoreType.DMA,
            pltpu.SemaphoreType.DMA,
            pltpu.SemaphoreType.DMA,
        ],
    )(_sc_head_body)


def _sc_head_body(a_hbm, b_hbm, eh_hbm, src_hbm, dst_hbm, t_hbm,
                  idx_s2, idx_d2, msg0, msg1, ra0, ra1, rb0, rb1,
                  si0, si1, se0, se1, sg0, sg1, sw0, sw1):
    idx_s = (idx_s2.at[0], idx_s2.at[1])
    idx_d = (idx_d2.at[0], idx_d2.at[1])
    msg = (msg0, msg1)
    rows_a = (ra0, ra1)
    rows_b = (rb0, rb1)
    sem_i = (si0, si1)
    sem_e = (se0, se1)
    sem_g = (sg0, sg1)
    sem_w = (sw0, sw1)

    cid = lax.axis_index("c")
    sid = lax.axis_index("s")
    ebase = cid * (_E // _NSC) + sid * _EPT

    def issue(g, k):
        base = ebase + g * _BE
        pltpu.async_copy(src_hbm.at[pl.ds(base, _BE)], idx_s[k], sem_i[k])
        pltpu.async_copy(dst_hbm.at[pl.ds(base, _BE)], idx_d[k], sem_i[k])
        pltpu.async_copy(eh_hbm.at[pl.ds(base, _BE)], msg[k], sem_e[k])
        pltpu.make_async_copy(src_hbm.at[pl.ds(base, _BE)], idx_s[k],
                              sem_i[k]).wait()
        pltpu.make_async_copy(dst_hbm.at[pl.ds(base, _BE)], idx_d[k],
                              sem_i[k]).wait()
        pltpu.async_copy(a_hbm.at[idx_s[k]], rows_a[k], sem_g[k])
        pltpu.async_copy(b_hbm.at[idx_d[k]], rows_b[k], sem_g[k])

    def stage(g, b, do_drain, do_issue_next):
        kc, kn = b, 1 - b
        base = ebase + g * _BE

        _when(do_drain, lambda: pltpu.make_async_copy(
            msg[kn], t_hbm.at[pl.ds(ebase + (g - 1) * _BE, _BE)],
            sem_w[kn]).wait())
        _when(do_issue_next, lambda: issue(g + 1, kn))
        pltpu.make_async_copy(eh_hbm.at[pl.ds(base, _BE)], msg[kc],
                              sem_e[kc]).wait()
        pltpu.make_async_copy(a_hbm.at[idx_s[kc]], rows_a[kc],
                              sem_g[kc]).wait()
        pltpu.make_async_copy(b_hbm.at[idx_d[kc]], rows_b[kc],
                              sem_g[kc]).wait()
        _relu_add_rows(msg[kc], (rows_a[kc], rows_b[kc]), _BE)
        pltpu.async_copy(msg[kc], t_hbm.at[pl.ds(base, _BE)], sem_w[kc])

    issue(0, 0)

    def pair(gg, _):
        g0 = gg * 2
        stage(g0, 0, g0 > 0, True)
        stage(g0 + 1, 1, True, True)
        return 0

    lax.fori_loop(0, _NCHUNK // 2, pair, 0)
    stage(_NCHUNK - 1, 0, True, False)
    pltpu.make_async_copy(
        msg[0], t_hbm.at[pl.ds(ebase + (_NCHUNK - 1) * _BE, _BE)],
        sem_w[0]).wait()


def kernel(x, edge_index, edge_attr, params):
    p = params
    layers = p['layers']
    src = edge_index[0]
    dst = edge_index[1]

    wcat = jnp.concatenate(
        [layers[0]['lin_w'], layers[1]['lin_w'], layers[2]['lin_w'],
         p['hw1'][2 * _H:]], axis=1)
    bcat = jnp.concatenate(
        [layers[0]['lin_b'], layers[1]['lin_b'], layers[2]['lin_b'], p['hb1']])
    e1, e2, e3, eh = _edge_pre(edge_attr, wcat, bcat)

    h = _mm_bias(x, p['in_w'], p['in_b'], relu=True, block_rows=1000)

    sc_layer = _make_sc_layer()
    for lp, e_l in zip(layers, (e1, e2, e3)):
        partials = sc_layer(h, e_l, src, dst)
        h = _node_mlp(h, partials[0], partials[1],
                      lp['w1'], lp['b1'], lp['w2'], lp['b2'],
                      lp['gamma'], lp['beta'])

    wab = jnp.concatenate([p['hw1'][:_H], p['hw1'][_H:2 * _H]], axis=1)
    a_tab, b_tab = _ab_tables(h, wab)
    t = _make_sc_head()(a_tab, b_tab, eh, src, dst)
    return _head_mlp(t, p['hw2'], p['hb2'], p['hw3'], p['hb3'])


# speedup vs baseline: 3.7535x; 1.0070x over previous
import functools

import jax
import jax.numpy as jnp
from jax import lax
from jax.experimental import pallas as pl
from jax.experimental.pallas import tpu as pltpu
from jax.experimental.pallas import tpu_sc as plsc

F32 = jnp.float32

_N = 10000
_E = 320000
_H = 128

_NSC = 2
_NT = 16
_EPT = _E // (_NSC * _NT)
_BE = 80
_NCHUNK = _EPT // _BE
_RPT = 624
_NTAIL = _N - _NT * _RPT


def _mm_bias(x, w, b, *, relu, block_rows):
    m, k = x.shape
    n = w.shape[1]

    def body(x_ref, w_ref, b_ref, o_ref):
        y = jnp.dot(x_ref[...], w_ref[...], preferred_element_type=F32)
        y = y + b_ref[...]
        if relu:
            y = jnp.maximum(y, 0.0)
        o_ref[...] = y

    return pl.pallas_call(
        body,
        grid=(m // block_rows,),
        in_specs=[
            pl.BlockSpec((block_rows, k), lambda i: (i, 0)),
            pl.BlockSpec((k, n), lambda i: (0, 0)),
            pl.BlockSpec((1, n), lambda i: (0, 0)),
        ],
        out_specs=pl.BlockSpec((block_rows, n), lambda i: (i, 0)),
        out_shape=jax.ShapeDtypeStruct((m, n), F32),
    )(x, w, b.reshape(1, -1))


def _edge_term(ea, w, b):
    be = 4000

    def body(ea_ref, w_ref, b_ref, o_ref):
        y = jnp.dot(ea_ref[...], w_ref[...], preferred_element_type=F32)
        o_ref[...] = y + b_ref[...]

    return pl.pallas_call(
        body,
        grid=(_E // be,),
        in_specs=[
            pl.BlockSpec((be, 16), lambda i: (i, 0)),
            pl.BlockSpec((16, 128), lambda i: (0, 0)),
            pl.BlockSpec((1, 128), lambda i: (0, 0)),
        ],
        out_specs=pl.BlockSpec((be, 128), lambda i: (i, 0)),
        out_shape=jax.ShapeDtypeStruct((_E, 128), F32),
    )(ea, w, b.reshape(1, -1))


def _node_mlp(h, p0, p1, w1, b1, w2, b2, gamma, beta):
    bn = 1000
    nb = _N // bn

    def body(h_ref, p0_ref, p1_ref, w1_ref, b1_ref, w2_ref, b2_ref,
             g_ref, be_ref, o_ref, y_scr, stat_scr):
        i = pl.program_id(0)

        z = h_ref[...] + p0_ref[...] + p1_ref[...]
        y = jnp.dot(z, w1_ref[...], preferred_element_type=F32) + b1_ref[...]
        y = jnp.maximum(y, 0.0)
        y = jnp.dot(y, w2_ref[...], preferred_element_type=F32) + b2_ref[...]
        y_scr[pl.ds(i * bn, bn), :] = y
        s = jnp.sum(y, axis=0, keepdims=True)
        ss = jnp.sum(y * y, axis=0, keepdims=True)

        @pl.when(i == 0)
        def _():
            stat_scr[0:1, :] = s
            stat_scr[1:2, :] = ss

        @pl.when(i > 0)
        def _():
            stat_scr[0:1, :] = stat_scr[0:1, :] + s
            stat_scr[1:2, :] = stat_scr[1:2, :] + ss

        @pl.when(i == nb - 1)
        def _():
            mean = stat_scr[0:1, :] * (1.0 / _N)
            var = stat_scr[1:2, :] * (1.0 / _N) - mean * mean
            scale = lax.rsqrt(var + 1e-5) * g_ref[...]
            shift = be_ref[...] - mean * scale
            for j in range(nb):
                yj = y_scr[pl.ds(j * bn, bn), :]
                o_ref[pl.ds(j * bn, bn), :] = jnp.maximum(
                    yj * scale + shift, 0.0)

    return pl.pallas_call(
        body,
        grid=(nb,),
        in_specs=[
            pl.BlockSpec((bn, _H), lambda i: (i, 0)),
            pl.BlockSpec((bn, _H), lambda i: (i, 0)),
            pl.BlockSpec((bn, _H), lambda i: (i, 0)),
            pl.BlockSpec((_H, 2 * _H), lambda i: (0, 0)),
            pl.BlockSpec((1, 2 * _H), lambda i: (0, 0)),
            pl.BlockSpec((2 * _H, _H), lambda i: (0, 0)),
            pl.BlockSpec((1, _H), lambda i: (0, 0)),
            pl.BlockSpec((1, _H), lambda i: (0, 0)),
            pl.BlockSpec((1, _H), lambda i: (0, 0)),
        ],
        out_specs=pl.BlockSpec((_N, _H), lambda i: (0, 0)),
        out_shape=jax.ShapeDtypeStruct((_N, _H), F32),
        scratch_shapes=[
            pltpu.VMEM((_N, _H), F32),
            pltpu.VMEM((8, _H), F32),
        ],
    )(h, p0, p1, w1, b1.reshape(1, -1), w2, b2.reshape(1, -1),
      gamma.reshape(1, -1), beta.reshape(1, -1))


def _ab_tables(h, wab):
    bn = 1000

    def body(h_ref, w_ref, oa, ob):
        y = jnp.dot(h_ref[...], w_ref[...], preferred_element_type=F32)
        oa[...] = y[:, 0:128]
        ob[...] = y[:, 128:256]

    return pl.pallas_call(
        body,
        grid=(_N // bn,),
        in_specs=[
            pl.BlockSpec((bn, _H), lambda i: (i, 0)),
            pl.BlockSpec((_H, 2 * _H), lambda i: (0, 0)),
        ],
        out_specs=[pl.BlockSpec((bn, _H), lambda i: (i, 0))] * 2,
        out_shape=[jax.ShapeDtypeStruct((_N, _H), F32)] * 2,
    )(h, wab)


def _head_mlp(t, w2, b2, w3, b3):
    be = 2000
    c = w3.shape[1]

    def body(t_ref, w2_ref, b2_ref, w3_ref, b3_ref, o_ref):
        y = jnp.dot(t_ref[...], w2_ref[...], preferred_element_type=F32)
        y = jnp.maximum(y + b2_ref[...], 0.0)
        o_ref[...] = jnp.dot(y, w3_ref[...], preferred_element_type=F32) + b3_ref[...]

    return pl.pallas_call(
        body,
        grid=(_E // be,),
        in_specs=[
            pl.BlockSpec((be, _H), lambda i: (i, 0)),
            pl.BlockSpec((_H, 64), lambda i: (0, 0)),
            pl.BlockSpec((1, 64), lambda i: (0, 0)),
            pl.BlockSpec((64, c), lambda i: (0, 0)),
            pl.BlockSpec((1, c), lambda i: (0, 0)),
        ],
        out_specs=pl.BlockSpec((be, c), lambda i: (i, 0)),
        out_shape=jax.ShapeDtypeStruct((_E, c), F32),
    )(t, w2, b2.reshape(1, -1), w3, b3.reshape(1, -1))


@functools.lru_cache(maxsize=None)
def _sc_mesh():
    return plsc.VectorSubcoreMesh(core_axis_name="c", subcore_axis_name="s",
                                  num_cores=_NSC, num_subcores=_NT)


def _relu_add_rows(msg, extra, nrows):
    @plsc.parallel_loop(0, nrows, 1, unroll=2)
    def _(r):
        for cc in range(_H // 16):
            sl = pl.ds(cc * 16, 16)
            v = msg[r, sl]
            for ex in extra:
                v = v + ex[r, sl]
            msg[r, sl] = jnp.maximum(v, 0.0)


def _when(cond, fn):
    if isinstance(cond, bool):
        if cond:
            fn()
    else:
        pl.when(cond)(fn)


@functools.lru_cache(maxsize=None)
def _make_sc_layer():
    return functools.partial(
        pl.kernel,
        out_type=jax.ShapeDtypeStruct((_NSC, _N, _H), F32),
        mesh=_sc_mesh(),
        scratch_types=[
            pltpu.VMEM((2, _BE), jnp.int32),
            pltpu.VMEM((2, _BE), jnp.int32),
            pltpu.VMEM((_BE, _H), F32),
            pltpu.VMEM((_BE, _H), F32),
            pltpu.VMEM((_BE, _H), F32),
            pltpu.VMEM((_BE, _H), F32),
            pltpu.VMEM((48, _H), F32),
            pltpu.SemaphoreType.DMA,
            pltpu.SemaphoreType.DMA,
            pltpu.SemaphoreType.DMA,
            pltpu.SemaphoreType.DMA,
            pltpu.SemaphoreType.DMA,
            pltpu.SemaphoreType.DMA,
            pltpu.SemaphoreType.DMA,
            pltpu.SemaphoreType.DMA,
            pltpu.VMEM_SHARED((_N, _H), F32),
        ],
    )(_sc_layer_body)


def _sc_layer_body(h_hbm, e_hbm, src_hbm, dst_hbm, out_hbm,
                   idx_s2, idx_d2, msg0, msg1, rows0, rows1, zbuf,
                   si0, si1, se0, se1, sg0, sg1, ss0, ss1, aggr):
    idx_s = (idx_s2.at[0], idx_s2.at[1])
    idx_d = (idx_d2.at[0], idx_d2.at[1])
    msg = (msg0, msg1)
    rows = (rows0, rows1)
    sem_i = (si0, si1)
    sem_e = (se0, se1)
    sem_g = (sg0, sg1)
    sem_s = (ss0, ss1)
    cid = lax.axis_index("c")
    sid = lax.axis_index("s")

    zero16 = jnp.zeros((16,), F32)

    def zrow(r, _):
        for cc in range(_H // 16):
            zbuf[r, pl.ds(cc * 16, 16)] = zero16
        return 0

    lax.fori_loop(0, 48, zrow, 0)
    row0 = sid * _RPT
    for j in range(_RPT // 48):
        pltpu.sync_copy(zbuf, aggr.at[pl.ds(row0 + j * 48, 48)])

    @pl.when(sid == _NT - 1)
    def _():
        pltpu.sync_copy(zbuf.at[pl.ds(0, _NTAIL)],
                        aggr.at[pl.ds(_NT * _RPT, _NTAIL)])

    plsc.subcore_barrier()

    ebase = cid * (_E // _NSC) + sid * _EPT

    def issue(g, k):
        base = ebase + g * _BE
        pltpu.async_copy(src_hbm.at[pl.ds(base, _BE)], idx_s[k], sem_i[k])
        pltpu.async_copy(dst_hbm.at[pl.ds(base, _BE)], idx_d[k], sem_i[k])
        pltpu.async_copy(e_hbm.at[pl.ds(base, _BE)], msg[k], sem_e[k])
        pltpu.make_async_copy(src_hbm.at[pl.ds(base, _BE)], idx_s[k],
                              sem_i[k]).wait()
        pltpu.async_copy(h_hbm.at[idx_s[k]], rows[k], sem_g[k])

    def stage(g, b, do_drain, do_issue_next):
        kc, kn = b, 1 - b
        base = ebase + g * _BE

        _when(do_drain, lambda: pltpu.make_async_copy(
            msg[kn], aggr.at[idx_d[kn]], sem_s[kn]).wait())
        _when(do_issue_next, lambda: issue(g + 1, kn))
        pltpu.make_async_copy(e_hbm.at[pl.ds(base, _BE)], msg[kc],
                              sem_e[kc]).wait()
        pltpu.make_async_copy(h_hbm.at[idx_s[kc]], rows[kc], sem_g[kc]).wait()
        _relu_add_rows(msg[kc], (rows[kc],), _BE)
        pltpu.make_async_copy(dst_hbm.at[pl.ds(base, _BE)], idx_d[kc],
                              sem_i[kc]).wait()
        pltpu.async_copy(msg[kc], aggr.at[idx_d[kc]], sem_s[kc], add=True)

    issue(0, 0)

    def pair(gg, _):
        g0 = gg * 2
        stage(g0, 0, g0 > 0, True)
        stage(g0 + 1, 1, True, True)
        return 0

    lax.fori_loop(0, _NCHUNK // 2, pair, 0)
    stage(_NCHUNK - 1, 0, True, False)
    pltpu.make_async_copy(msg[0], aggr.at[idx_d[0]], sem_s[0]).wait()
    plsc.subcore_barrier()
    pltpu.sync_copy(aggr.at[pl.ds(row0, _RPT)],
                    out_hbm.at[cid, pl.ds(row0, _RPT)])

    @pl.when(sid == _NT - 1)
    def _():
        pltpu.sync_copy(aggr.at[pl.ds(_NT * _RPT, _NTAIL)],
                        out_hbm.at[cid, pl.ds(_NT * _RPT, _NTAIL)])


@functools.lru_cache(maxsize=None)
def _make_sc_head():
    return functools.partial(
        pl.kernel,
        out_type=jax.ShapeDtypeStruct((_E, _H), F32),
        mesh=_sc_mesh(),
        scratch_types=[
            pltpu.VMEM((2, _BE), jnp.int32),
            pltpu.VMEM((2, _BE), jnp.int32),
            pltpu.VMEM((_BE, _H), F32),
            pltpu.VMEM((_BE, _H), F32),
            pltpu.VMEM((_BE, _H), F32),
            pltpu.VMEM((_BE, _H), F32),
            pltpu.VMEM((_BE, _H), F32),
            pltpu.VMEM((_BE, _H), F32),
            pltpu.SemaphoreType.DMA,
            pltpu.SemaphoreType.DMA,
            pltpu.SemaphoreType.DMA,
            pltpu.SemaphoreType.DMA,
            pltpu.SemaphoreType.DMA,
            pltpu.SemaphoreType.DMA,
            pltpu.SemaphoreType.DMA,
            pltpu.SemaphoreType.DMA,
        ],
    )(_sc_head_body)


def _sc_head_body(a_hbm, b_hbm, eh_hbm, src_hbm, dst_hbm, t_hbm,
                  idx_s2, idx_d2, msg0, msg1, ra0, ra1, rb0, rb1,
                  si0, si1, se0, se1, sg0, sg1, sw0, sw1):
    idx_s = (idx_s2.at[0], idx_s2.at[1])
    idx_d = (idx_d2.at[0], idx_d2.at[1])
    msg = (msg0, msg1)
    rows_a = (ra0, ra1)
    rows_b = (rb0, rb1)
    sem_i = (si0, si1)
    sem_e = (se0, se1)
    sem_g = (sg0, sg1)
    sem_w = (sw0, sw1)

    cid = lax.axis_index("c")
    sid = lax.axis_index("s")
    ebase = cid * (_E // _NSC) + sid * _EPT

    def issue(g, k):
        base = ebase + g * _BE
        pltpu.async_copy(src_hbm.at[pl.ds(base, _BE)], idx_s[k], sem_i[k])
        pltpu.async_copy(dst_hbm.at[pl.ds(base, _BE)], idx_d[k], sem_i[k])
        pltpu.async_copy(eh_hbm.at[pl.ds(base, _BE)], msg[k], sem_e[k])
        pltpu.make_async_copy(src_hbm.at[pl.ds(base, _BE)], idx_s[k],
                              sem_i[k]).wait()
        pltpu.make_async_copy(dst_hbm.at[pl.ds(base, _BE)], idx_d[k],
                              sem_i[k]).wait()
        pltpu.async_copy(a_hbm.at[idx_s[k]], rows_a[k], sem_g[k])
        pltpu.async_copy(b_hbm.at[idx_d[k]], rows_b[k], sem_g[k])

    def stage(g, b, do_drain, do_issue_next):
        kc, kn = b, 1 - b
        base = ebase + g * _BE

        _when(do_drain, lambda: pltpu.make_async_copy(
            msg[kn], t_hbm.at[pl.ds(ebase + (g - 1) * _BE, _BE)],
            sem_w[kn]).wait())
        _when(do_issue_next, lambda: issue(g + 1, kn))
        pltpu.make_async_copy(eh_hbm.at[pl.ds(base, _BE)], msg[kc],
                              sem_e[kc]).wait()
        pltpu.make_async_copy(a_hbm.at[idx_s[kc]], rows_a[kc],
                              sem_g[kc]).wait()
        pltpu.make_async_copy(b_hbm.at[idx_d[kc]], rows_b[kc],
                              sem_g[kc]).wait()
        _relu_add_rows(msg[kc], (rows_a[kc], rows_b[kc]), _BE)
        pltpu.async_copy(msg[kc], t_hbm.at[pl.ds(base, _BE)], sem_w[kc])

    issue(0, 0)

    def pair(gg, _):
        g0 = gg * 2
        stage(g0, 0, g0 > 0, True)
        stage(g0 + 1, 1, True, True)
        return 0

    lax.fori_loop(0, _NCHUNK // 2, pair, 0)
    stage(_NCHUNK - 1, 0, True, False)
    pltpu.make_async_copy(
        msg[0], t_hbm.at[pl.ds(ebase + (_NCHUNK - 1) * _BE, _BE)],
        sem_w[0]).wait()


def kernel(x, edge_index, edge_attr, params):
    p = params
    layers = p['layers']
    src = edge_index[0]
    dst = edge_index[1]

    e1 = _edge_term(edge_attr, layers[0]['lin_w'], layers[0]['lin_b'])
    e2 = _edge_term(edge_attr, layers[1]['lin_w'], layers[1]['lin_b'])
    e3 = _edge_term(edge_attr, layers[2]['lin_w'], layers[2]['lin_b'])
    eh = _edge_term(edge_attr, p['hw1'][2 * _H:], p['hb1'])

    h = _mm_bias(x, p['in_w'], p['in_b'], relu=True, block_rows=1000)

    sc_layer = _make_sc_layer()
    for lp, e_l in zip(layers, (e1, e2, e3)):
        partials = sc_layer(h, e_l, src, dst)
        h = _node_mlp(h, partials[0], partials[1],
                      lp['w1'], lp['b1'], lp['w2'], lp['b2'],
                      lp['gamma'], lp['beta'])

    wab = jnp.concatenate([p['hw1'][:_H], p['hw1'][_H:2 * _H]], axis=1)
    a_tab, b_tab = _ab_tables(h, wab)
    t = _make_sc_head()(a_tab, b_tab, eh, src, dst)
    return _head_mlp(t, p['hw2'], p['hb2'], p['hw3'], p['hb3'])


# speedup vs baseline: 3.7659x; 1.0033x over previous
import functools

import jax
import jax.numpy as jnp
from jax import lax
from jax.experimental import pallas as pl
from jax.experimental.pallas import tpu as pltpu
from jax.experimental.pallas import tpu_sc as plsc

F32 = jnp.float32

_N = 10000
_E = 320000
_H = 128

_NSC = 2
_NT = 16
_EPT = _E // (_NSC * _NT)
_BE = 80
_NCHUNK = _EPT // _BE
_RPT = 624
_NTAIL = _N - _NT * _RPT


def _mm_bias(x, w, b, *, relu, block_rows):
    m, k = x.shape
    n = w.shape[1]

    def body(x_ref, w_ref, b_ref, o_ref):
        y = jnp.dot(x_ref[...], w_ref[...], preferred_element_type=F32)
        y = y + b_ref[...]
        if relu:
            y = jnp.maximum(y, 0.0)
        o_ref[...] = y

    return pl.pallas_call(
        body,
        grid=(m // block_rows,),
        in_specs=[
            pl.BlockSpec((block_rows, k), lambda i: (i, 0)),
            pl.BlockSpec((k, n), lambda i: (0, 0)),
            pl.BlockSpec((1, n), lambda i: (0, 0)),
        ],
        out_specs=pl.BlockSpec((block_rows, n), lambda i: (i, 0)),
        out_shape=jax.ShapeDtypeStruct((m, n), F32),
    )(x, w, b.reshape(1, -1))


def _edge_term(ea, w, b):
    be = 4000

    def body(ea_ref, w_ref, b_ref, o_ref):
        y = jnp.dot(ea_ref[...], w_ref[...], preferred_element_type=F32)
        o_ref[...] = y + b_ref[...]

    return pl.pallas_call(
        body,
        grid=(_E // be,),
        in_specs=[
            pl.BlockSpec((be, 16), lambda i: (i, 0)),
            pl.BlockSpec((16, 128), lambda i: (0, 0)),
            pl.BlockSpec((1, 128), lambda i: (0, 0)),
        ],
        out_specs=pl.BlockSpec((be, 128), lambda i: (i, 0)),
        out_shape=jax.ShapeDtypeStruct((_E, 128), F32),
    )(ea, w, b.reshape(1, -1))


def _node_mlp(h, p0, p1, w1, b1, w2, b2, gamma, beta):
    bn = 1000
    nb = _N // bn

    def body(h_ref, p0_ref, p1_ref, w1_ref, b1_ref, w2_ref, b2_ref,
             g_ref, be_ref, o_ref, y_scr, stat_scr):
        i = pl.program_id(0)

        z = h_ref[...] + p0_ref[...] + p1_ref[...]
        y = jnp.dot(z, w1_ref[...], preferred_element_type=F32) + b1_ref[...]
        y = jnp.maximum(y, 0.0)
        y = jnp.dot(y, w2_ref[...], preferred_element_type=F32) + b2_ref[...]
        y_scr[pl.ds(i * bn, bn), :] = y
        s = jnp.sum(y, axis=0, keepdims=True)
        ss = jnp.sum(y * y, axis=0, keepdims=True)

        @pl.when(i == 0)
        def _():
            stat_scr[0:1, :] = s
            stat_scr[1:2, :] = ss

        @pl.when(i > 0)
        def _():
            stat_scr[0:1, :] = stat_scr[0:1, :] + s
            stat_scr[1:2, :] = stat_scr[1:2, :] + ss

        @pl.when(i == nb - 1)
        def _():
            mean = stat_scr[0:1, :] * (1.0 / _N)
            var = stat_scr[1:2, :] * (1.0 / _N) - mean * mean
            scale = lax.rsqrt(var + 1e-5) * g_ref[...]
            shift = be_ref[...] - mean * scale
            for j in range(nb):
                yj = y_scr[pl.ds(j * bn, bn), :]
                o_ref[pl.ds(j * bn, bn), :] = jnp.maximum(
                    yj * scale + shift, 0.0)

    return pl.pallas_call(
        body,
        grid=(nb,),
        in_specs=[
            pl.BlockSpec((bn, _H), lambda i: (i, 0)),
            pl.BlockSpec((bn, _H), lambda i: (i, 0)),
            pl.BlockSpec((bn, _H), lambda i: (i, 0)),
            pl.BlockSpec((_H, 2 * _H), lambda i: (0, 0)),
            pl.BlockSpec((1, 2 * _H), lambda i: (0, 0)),
            pl.BlockSpec((2 * _H, _H), lambda i: (0, 0)),
            pl.BlockSpec((1, _H), lambda i: (0, 0)),
            pl.BlockSpec((1, _H), lambda i: (0, 0)),
            pl.BlockSpec((1, _H), lambda i: (0, 0)),
        ],
        out_specs=pl.BlockSpec((_N, _H), lambda i: (0, 0)),
        out_shape=jax.ShapeDtypeStruct((_N, _H), F32),
        scratch_shapes=[
            pltpu.VMEM((_N, _H), F32),
            pltpu.VMEM((8, _H), F32),
        ],
    )(h, p0, p1, w1, b1.reshape(1, -1), w2, b2.reshape(1, -1),
      gamma.reshape(1, -1), beta.reshape(1, -1))


def _ab_tables(h, wab):
    bn = 1000

    def body(h_ref, w_ref, oa, ob):
        y = jnp.dot(h_ref[...], w_ref[...], preferred_element_type=F32)
        oa[...] = y[:, 0:128]
        ob[...] = y[:, 128:256]

    return pl.pallas_call(
        body,
        grid=(_N // bn,),
        in_specs=[
            pl.BlockSpec((bn, _H), lambda i: (i, 0)),
            pl.BlockSpec((_H, 2 * _H), lambda i: (0, 0)),
        ],
        out_specs=[pl.BlockSpec((bn, _H), lambda i: (i, 0))] * 2,
        out_shape=[jax.ShapeDtypeStruct((_N, _H), F32)] * 2,
    )(h, wab)


def _head_mlp(t, w2, b2, w3, b3):
    be = 2000
    c = w3.shape[1]

    def body(t_ref, w2_ref, b2_ref, w3_ref, b3_ref, o_ref):
        y = jnp.dot(t_ref[...], w2_ref[...], preferred_element_type=F32)
        y = jnp.maximum(y + b2_ref[...], 0.0)
        o_ref[...] = jnp.dot(y, w3_ref[...], preferred_element_type=F32) + b3_ref[...]

    return pl.pallas_call(
        body,
        grid=(_E // be,),
        in_specs=[
            pl.BlockSpec((be, _H), lambda i: (i, 0)),
            pl.BlockSpec((_H, 64), lambda i: (0, 0)),
            pl.BlockSpec((1, 64), lambda i: (0, 0)),
            pl.BlockSpec((64, c), lambda i: (0, 0)),
            pl.BlockSpec((1, c), lambda i: (0, 0)),
        ],
        out_specs=pl.BlockSpec((be, c), lambda i: (i, 0)),
        out_shape=jax.ShapeDtypeStruct((_E, c), F32),
    )(t, w2, b2.reshape(1, -1), w3, b3.reshape(1, -1))


@functools.lru_cache(maxsize=None)
def _sc_mesh():
    return plsc.VectorSubcoreMesh(core_axis_name="c", subcore_axis_name="s",
                                  num_cores=_NSC, num_subcores=_NT)


def _relu_add_rows(msg, extra, nrows):
    def row(r, _):
        for cc in range(_H // 16):
            sl = pl.ds(cc * 16, 16)
            v = msg[r, sl]
            for ex in extra:
                v = v + ex[r, sl]
            msg[r, sl] = jnp.maximum(v, 0.0)
        return 0

    lax.fori_loop(0, nrows, row, 0)


def _when(cond, fn):
    if isinstance(cond, bool):
        if cond:
            fn()
    else:
        pl.when(cond)(fn)


@functools.lru_cache(maxsize=None)
def _make_sc_layer():
    return functools.partial(
        pl.kernel,
        out_type=jax.ShapeDtypeStruct((_NSC, _N, _H), F32),
        mesh=_sc_mesh(),
        scratch_types=[
            pltpu.VMEM((2, _BE), jnp.int32),
            pltpu.VMEM((2, _BE), jnp.int32),
            pltpu.VMEM((_BE, _H), F32),
            pltpu.VMEM((_BE, _H), F32),
            pltpu.VMEM((_BE, _H), F32),
            pltpu.VMEM((_BE, _H), F32),
            pltpu.VMEM((48, _H), F32),
            pltpu.SemaphoreType.DMA,
            pltpu.SemaphoreType.DMA,
            pltpu.SemaphoreType.DMA,
            pltpu.SemaphoreType.DMA,
            pltpu.SemaphoreType.DMA,
            pltpu.SemaphoreType.DMA,
            pltpu.SemaphoreType.DMA,
            pltpu.SemaphoreType.DMA,
            pltpu.VMEM_SHARED((_N, _H), F32),
        ],
    )(_sc_layer_body)


def _sc_layer_body(h_hbm, e_hbm, src_hbm, dst_hbm, out_hbm,
                   idx_s2, idx_d2, msg0, msg1, rows0, rows1, zbuf,
                   si0, si1, se0, se1, sg0, sg1, ss0, ss1, aggr):
    idx_s = (idx_s2.at[0], idx_s2.at[1])
    idx_d = (idx_d2.at[0], idx_d2.at[1])
    msg = (msg0, msg1)
    rows = (rows0, rows1)
    sem_i = (si0, si1)
    sem_e = (se0, se1)
    sem_g = (sg0, sg1)
    sem_s = (ss0, ss1)
    cid = lax.axis_index("c")
    sid = lax.axis_index("s")

    zero16 = jnp.zeros((16,), F32)

    def zrow(r, _):
        for cc in range(_H // 16):
            zbuf[r, pl.ds(cc * 16, 16)] = zero16
        return 0

    lax.fori_loop(0, 48, zrow, 0)
    row0 = sid * _RPT
    for j in range(_RPT // 48):
        pltpu.sync_copy(zbuf, aggr.at[pl.ds(row0 + j * 48, 48)])

    @pl.when(sid == _NT - 1)
    def _():
        pltpu.sync_copy(zbuf.at[pl.ds(0, _NTAIL)],
                        aggr.at[pl.ds(_NT * _RPT, _NTAIL)])

    plsc.subcore_barrier()

    ebase = cid * (_E // _NSC) + sid * _EPT

    def issue(g, k):
        base = ebase + g * _BE
        pltpu.async_copy(src_hbm.at[pl.ds(base, _BE)], idx_s[k], sem_i[k])
        pltpu.async_copy(dst_hbm.at[pl.ds(base, _BE)], idx_d[k], sem_i[k])
        pltpu.async_copy(e_hbm.at[pl.ds(base, _BE)], msg[k], sem_e[k])
        pltpu.make_async_copy(src_hbm.at[pl.ds(base, _BE)], idx_s[k],
                              sem_i[k]).wait()
        pltpu.async_copy(h_hbm.at[idx_s[k]], rows[k], sem_g[k])

    def stage(g, b, do_drain, do_issue_next):
        kc, kn = b, 1 - b
        base = ebase + g * _BE

        _when(do_drain, lambda: pltpu.make_async_copy(
            msg[kn], aggr.at[idx_d[kn]], sem_s[kn]).wait())
        _when(do_issue_next, lambda: issue(g + 1, kn))
        pltpu.make_async_copy(e_hbm.at[pl.ds(base, _BE)], msg[kc],
                              sem_e[kc]).wait()
        pltpu.make_async_copy(h_hbm.at[idx_s[kc]], rows[kc], sem_g[kc]).wait()
        _relu_add_rows(msg[kc], (rows[kc],), _BE)
        pltpu.make_async_copy(dst_hbm.at[pl.ds(base, _BE)], idx_d[kc],
                              sem_i[kc]).wait()
        pltpu.async_copy(msg[kc], aggr.at[idx_d[kc]], sem_s[kc], add=True)

    issue(0, 0)

    def pair(gg, _):
        g0 = gg * 2
        stage(g0, 0, g0 > 0, True)
        stage(g0 + 1, 1, True, True)
        return 0

    lax.fori_loop(0, _NCHUNK // 2, pair, 0)
    stage(_NCHUNK - 1, 0, True, False)
    pltpu.make_async_copy(msg[0], aggr.at[idx_d[0]], sem_s[0]).wait()
    plsc.subcore_barrier()
    pltpu.sync_copy(aggr.at[pl.ds(row0, _RPT)],
                    out_hbm.at[cid, pl.ds(row0, _RPT)])

    @pl.when(sid == _NT - 1)
    def _():
        pltpu.sync_copy(aggr.at[pl.ds(_NT * _RPT, _NTAIL)],
                        out_hbm.at[cid, pl.ds(_NT * _RPT, _NTAIL)])


@functools.lru_cache(maxsize=None)
def _make_sc_head():
    return functools.partial(
        pl.kernel,
        out_type=jax.ShapeDtypeStruct((_E, _H), F32),
        mesh=_sc_mesh(),
        scratch_types=[
            pltpu.VMEM((2, _BE), jnp.int32),
            pltpu.VMEM((2, _BE), jnp.int32),
            pltpu.VMEM((_BE, _H), F32),
            pltpu.VMEM((_BE, _H), F32),
            pltpu.VMEM((_BE, _H), F32),
            pltpu.VMEM((_BE, _H), F32),
            pltpu.VMEM((_BE, _H), F32),
            pltpu.VMEM((_BE, _H), F32),
            pltpu.SemaphoreType.DMA,
            pltpu.SemaphoreType.DMA,
            pltpu.SemaphoreType.DMA,
            pltpu.SemaphoreType.DMA,
            pltpu.SemaphoreType.DMA,
            pltpu.SemaphoreType.DMA,
            pltpu.SemaphoreType.DMA,
            pltpu.SemaphoreType.DMA,
        ],
    )(_sc_head_body)


def _sc_head_body(a_hbm, b_hbm, eh_hbm, src_hbm, dst_hbm, t_hbm,
                  idx_s2, idx_d2, msg0, msg1, ra0, ra1, rb0, rb1,
                  si0, si1, se0, se1, sg0, sg1, sw0, sw1):
    idx_s = (idx_s2.at[0], idx_s2.at[1])
    idx_d = (idx_d2.at[0], idx_d2.at[1])
    msg = (msg0, msg1)
    rows_a = (ra0, ra1)
    rows_b = (rb0, rb1)
    sem_i = (si0, si1)
    sem_e = (se0, se1)
    sem_g = (sg0, sg1)
    sem_w = (sw0, sw1)

    cid = lax.axis_index("c")
    sid = lax.axis_index("s")
    ebase = cid * (_E // _NSC) + sid * _EPT

    def issue(g, k):
        base = ebase + g * _BE
        pltpu.async_copy(src_hbm.at[pl.ds(base, _BE)], idx_s[k], sem_i[k])
        pltpu.async_copy(dst_hbm.at[pl.ds(base, _BE)], idx_d[k], sem_i[k])
        pltpu.async_copy(eh_hbm.at[pl.ds(base, _BE)], msg[k], sem_e[k])
        pltpu.make_async_copy(src_hbm.at[pl.ds(base, _BE)], idx_s[k],
                              sem_i[k]).wait()
        pltpu.make_async_copy(dst_hbm.at[pl.ds(base, _BE)], idx_d[k],
                              sem_i[k]).wait()
        pltpu.async_copy(a_hbm.at[idx_s[k]], rows_a[k], sem_g[k])
        pltpu.async_copy(b_hbm.at[idx_d[k]], rows_b[k], sem_g[k])

    def stage(g, b, do_drain, do_issue_next):
        kc, kn = b, 1 - b
        base = ebase + g * _BE

        _when(do_drain, lambda: pltpu.make_async_copy(
            msg[kn], t_hbm.at[pl.ds(ebase + (g - 1) * _BE, _BE)],
            sem_w[kn]).wait())
        _when(do_issue_next, lambda: issue(g + 1, kn))
        pltpu.make_async_copy(eh_hbm.at[pl.ds(base, _BE)], msg[kc],
                              sem_e[kc]).wait()
        pltpu.make_async_copy(a_hbm.at[idx_s[kc]], rows_a[kc],
                              sem_g[kc]).wait()
        pltpu.make_async_copy(b_hbm.at[idx_d[kc]], rows_b[kc],
                              sem_g[kc]).wait()
        _relu_add_rows(msg[kc], (rows_a[kc], rows_b[kc]), _BE)
        pltpu.async_copy(msg[kc], t_hbm.at[pl.ds(base, _BE)], sem_w[kc])

    issue(0, 0)

    def pair(gg, _):
        g0 = gg * 2
        stage(g0, 0, g0 > 0, True)
        stage(g0 + 1, 1, True, True)
        return 0

    lax.fori_loop(0, _NCHUNK // 2, pair, 0)
    stage(_NCHUNK - 1, 0, True, False)
    pltpu.make_async_copy(
        msg[0], t_hbm.at[pl.ds(ebase + (_NCHUNK - 1) * _BE, _BE)],
        sem_w[0]).wait()


def kernel(x, edge_index, edge_attr, params):
    p = params
    layers = p['layers']
    src = edge_index[0]
    dst = edge_index[1]

    e1 = _edge_term(edge_attr, layers[0]['lin_w'], layers[0]['lin_b'])
    e2 = _edge_term(edge_attr, layers[1]['lin_w'], layers[1]['lin_b'])
    e3 = _edge_term(edge_attr, layers[2]['lin_w'], layers[2]['lin_b'])
    eh = _edge_term(edge_attr, p['hw1'][2 * _H:], p['hb1'])

    h = _mm_bias(x, p['in_w'], p['in_b'], relu=True, block_rows=1000)

    sc_layer = _make_sc_layer()
    for lp, e_l in zip(layers, (e1, e2, e3)):
        partials = sc_layer(h, e_l, src, dst)
        h = _node_mlp(h, partials[0], partials[1],
                      lp['w1'], lp['b1'], lp['w2'], lp['b2'],
                      lp['gamma'], lp['beta'])

    wab = jnp.concatenate([p['hw1'][:_H], p['hw1'][_H:2 * _H]], axis=1)
    a_tab, b_tab = _ab_tables(h, wab)
    t = _make_sc_head()(a_tab, b_tab, eh, src, dst)
    return _head_mlp(t, p['hw2'], p['hb2'], p['hw3'], p['hb3'])
